# baseline (speedup 1.0000x reference)
import functools

import jax
import jax.numpy as jnp
from jax import lax
from jax.experimental import pallas as pl
from jax.experimental.pallas import tpu as pltpu
from jax.experimental.pallas import tpu_sc as plsc

D = 128
H = 8
DH = D // H
NL = 50000
HOPS = 2


def _pad_rows(x, tm):
    n = x.shape[0]
    npad = ((n + tm - 1) // tm) * tm
    if npad != n:
        x = jnp.pad(x, ((0, npad - n),) + ((0, 0),) * (x.ndim - 1))
    return x


_SC_NW = 32
_SC_CH = 128


@functools.cache
def _sc_gather_call(Ep, C, dtype_name):
    chunks = Ep // (_SC_NW * _SC_CH)
    dtype = jnp.dtype(dtype_name)
    mesh = plsc.VectorSubcoreMesh(core_axis_name="c", subcore_axis_name="s")
    cparams = (pltpu.CompilerParams(use_tc_tiling_on_sc=False)
               if (C % 128 or dtype == jnp.bfloat16) else None)

    @functools.partial(
        pl.kernel, mesh=mesh,
        out_type=jax.ShapeDtypeStruct((Ep, C), dtype),
        compiler_params=cparams,
        scratch_types=[
            pltpu.VMEM((chunks, _SC_CH), jnp.int32),
            pltpu.VMEM((_SC_CH, C), dtype),
            pltpu.SemaphoreType.DMA,
        ],
    )
    def gk(table_hbm, idx_hbm, out_hbm, idx_v, buf0, sem0):
        wid = lax.axis_index("s") * 2 + lax.axis_index("c")
        pltpu.sync_copy(idx_hbm.at[wid], idx_v)
        base = wid * chunks * _SC_CH

        def body(j, carry):
            pltpu.async_copy(table_hbm.at[idx_v.at[j]], buf0, sem0).wait()
            pltpu.sync_copy(buf0, out_hbm.at[pl.ds(base + j * _SC_CH, _SC_CH)])
            return carry

        lax.fori_loop(0, chunks, body, 0)

    return gk


def _sc_gather(table, idx):
    E = idx.shape[0]
    C = table.shape[1]
    unit = _SC_NW * _SC_CH
    Ep = ((E + unit - 1) // unit) * unit
    chunks = Ep // unit
    idxp = jnp.pad(idx.astype(jnp.int32), (0, Ep - E)).reshape(_SC_NW, chunks, _SC_CH)
    out = _sc_gather_call(Ep, C, str(table.dtype))(table, idxp)
    return out[:E]


_NH_CH = 8192


@functools.cache
def _sc_nhop_call(Ep_p, Es_p):
    n_pad = NL + 16
    mesh = plsc.VectorSubcoreMesh(core_axis_name="c", subcore_axis_name="s")

    @functools.partial(
        pl.kernel, mesh=mesh,
        out_type=(jax.ShapeDtypeStruct((Ep_p,), jnp.int32),
                  jax.ShapeDtypeStruct((Es_p,), jnp.int32)),
        compiler_params=pltpu.CompilerParams(needs_layout_passes=False),
        scratch_types=[
            pltpu.VMEM((n_pad,), jnp.int32),
            pltpu.VMEM((_NH_CH,), jnp.int32),
            pltpu.VMEM((_NH_CH,), jnp.int32),
            pltpu.VMEM((_NH_CH,), jnp.int32),
        ],
    )
    def nk(pdst_hbm, psrc_hbm, sdst_hbm, ssrc_hbm, pout_hbm, sout_hbm,
           m_v, dst_v, src_v, out_v):
        wid = lax.axis_index("s") * 2 + lax.axis_index("c")

        iota16 = jnp.arange(16, dtype=jnp.int32)

        def run(dst_hbm, src_hbm, out_hbm, e_pad):
            nchunk = e_pad // _NH_CH

            def init(k, c):
                m_v[pl.ds(k * 16, 16)] = iota16 + k * 16
                return c

            lax.fori_loop(0, n_pad // 16, init, 0)

            def chunk(j, c):
                pltpu.sync_copy(dst_hbm.at[pl.ds(j * _NH_CH, _NH_CH)], dst_v)
                pltpu.sync_copy(src_hbm.at[pl.ds(j * _NH_CH, _NH_CH)], src_v)

                def upd16(k, c2):
                    dv = dst_v[pl.ds(k * 16, 16)]
                    sv = src_v[pl.ds(k * 16, 16)]
                    p = (iota16 << 16) | sv
                    plsc.store_scatter(m_v, [dv], p)

                    def rnd(r, c3):
                        gw = plsc.load_gather(m_v, [dv])
                        plsc.store_scatter(m_v, [dv], p, mask=p > gw)
                        return c3

                    lax.fori_loop(0, 2, rnd, 0)
                    return c2

                lax.fori_loop(0, _NH_CH // 16, upd16, 0)
                return c

            lax.fori_loop(0, nchunk, chunk, 0)

            def chunk2(j, c):
                pltpu.sync_copy(src_hbm.at[pl.ds(j * _NH_CH, _NH_CH)], src_v)

                def gat(k, c2):
                    idx = src_v[pl.ds(k * 16, 16)]
                    g = plsc.load_gather(m_v, [idx])
                    out_v[pl.ds(k * 16, 16)] = g & 0xFFFF
                    return c2

                lax.fori_loop(0, _NH_CH // 16, gat, 0, unroll=4)
                pltpu.sync_copy(out_v, out_hbm.at[pl.ds(j * _NH_CH, _NH_CH)])
                return c

            lax.fori_loop(0, nchunk, chunk2, 0)

        @pl.when(wid == 0)
        def _():
            run(pdst_hbm, psrc_hbm, pout_hbm, Ep_p)

        @pl.when(wid == 1)
        def _():
            run(sdst_hbm, ssrc_hbm, sout_hbm, Es_p)

    return nk


def _sc_nhop(pred_ei, succ_ei):
    ep, es = pred_ei.shape[1], succ_ei.shape[1]
    epp = ((ep + _NH_CH - 1) // _NH_CH) * _NH_CH
    esp = ((es + _NH_CH - 1) // _NH_CH) * _NH_CH

    def prep(ei, n_pad_to):
        dst = jnp.pad(ei[1].astype(jnp.int32), (0, n_pad_to - ei.shape[1]),
                      constant_values=NL)
        src = jnp.pad(ei[0].astype(jnp.int32), (0, n_pad_to - ei.shape[1]))
        return dst, src

    pdst, psrc = prep(pred_ei, epp)
    sdst, ssrc = prep(succ_ei, esp)
    pout, sout = _sc_nhop_call(epp, esp)(pdst, psrc, sdst, ssrc)
    return pout[:ep], sout[:es]


def _mlp1_body(x_ref, w1_ref, b1_ref, w2_ref, b2_ref, o_ref):
    h = jnp.maximum(
        jnp.dot(x_ref[...], w1_ref[...], preferred_element_type=jnp.float32)
        + b1_ref[...], 0.0)
    o_ref[...] = jnp.dot(h, w2_ref[...], preferred_element_type=jnp.float32) + b2_ref[...]


def _mlp1(x, w1, b1, w2, b2, tm=1024):
    n, din = x.shape
    xp = _pad_rows(x, tm)
    g = xp.shape[0] // tm
    out = pl.pallas_call(
        _mlp1_body,
        grid=(g,),
        in_specs=[
            pl.BlockSpec((tm, din), lambda i: (i, 0)),
            pl.BlockSpec((din, D), lambda i: (0, 0)),
            pl.BlockSpec((1, D), lambda i: (0, 0)),
            pl.BlockSpec((D, D), lambda i: (0, 0)),
            pl.BlockSpec((1, D), lambda i: (0, 0)),
        ],
        out_specs=pl.BlockSpec((tm, D), lambda i: (i, 0)),
        out_shape=jax.ShapeDtypeStruct((xp.shape[0], D), jnp.float32),
    )(xp, w1, b1.reshape(1, D), w2, b2.reshape(1, D))
    return out[:n]


def _mlp2_body(x_ref, w1_ref, b1_ref, wa_ref, ba_ref, wb_ref, bb_ref, oa_ref, ob_ref):
    h = jnp.maximum(
        jnp.dot(x_ref[...], w1_ref[...], preferred_element_type=jnp.float32)
        + b1_ref[...], 0.0)
    oa_ref[...] = jnp.dot(h, wa_ref[...], preferred_element_type=jnp.float32) + ba_ref[...]
    ob_ref[...] = jnp.dot(h, wb_ref[...], preferred_element_type=jnp.float32) + bb_ref[...]


def _mlp2(x, w1, b1, wa, ba, wb, bb, tm=1024):
    n, din = x.shape
    xp = _pad_rows(x, tm)
    g = xp.shape[0] // tm
    oa, ob = pl.pallas_call(
        _mlp2_body,
        grid=(g,),
        in_specs=[
            pl.BlockSpec((tm, din), lambda i: (i, 0)),
            pl.BlockSpec((din, D), lambda i: (0, 0)),
            pl.BlockSpec((1, D), lambda i: (0, 0)),
            pl.BlockSpec((D, D), lambda i: (0, 0)),
            pl.BlockSpec((1, D), lambda i: (0, 0)),
            pl.BlockSpec((D, D), lambda i: (0, 0)),
            pl.BlockSpec((1, D), lambda i: (0, 0)),
        ],
        out_specs=[pl.BlockSpec((tm, D), lambda i: (i, 0))] * 2,
        out_shape=[jax.ShapeDtypeStruct((xp.shape[0], D), jnp.float32)] * 2,
    )(xp, w1, b1.reshape(1, D), wa, ba.reshape(1, D), wb, bb.reshape(1, D))
    return oa[:n], ob[:n]


def _matmul1_body(x_ref, w_ref, o_ref):
    o_ref[...] = jnp.dot(x_ref[...], w_ref[...], preferred_element_type=jnp.float32)


def _matmul1(x, w, tm=1024):
    n = x.shape[0]
    xp = _pad_rows(x, tm)
    g = xp.shape[0] // tm
    out = pl.pallas_call(
        _matmul1_body,
        grid=(g,),
        in_specs=[
            pl.BlockSpec((tm, D), lambda i: (i, 0)),
            pl.BlockSpec((D, D), lambda i: (0, 0)),
        ],
        out_specs=pl.BlockSpec((tm, D), lambda i: (i, 0)),
        out_shape=jax.ShapeDtypeStruct((xp.shape[0], D), jnp.float32),
    )(xp, w)
    return out[:n]


def _matmulkv_body(x_ref, wa_ref, wb_ref, o_ref):
    x = x_ref[...]
    o_ref[:, :D] = jnp.dot(x, wa_ref[...], preferred_element_type=jnp.float32)
    o_ref[:, D:] = jnp.dot(x, wb_ref[...], preferred_element_type=jnp.float32)


def _matmulkv(x, wa, wb, tm=1024):
    n = x.shape[0]
    xp = _pad_rows(x, tm)
    g = xp.shape[0] // tm
    out = pl.pallas_call(
        _matmulkv_body,
        grid=(g,),
        in_specs=[
            pl.BlockSpec((tm, D), lambda i: (i, 0)),
            pl.BlockSpec((D, D), lambda i: (0, 0)),
            pl.BlockSpec((D, D), lambda i: (0, 0)),
        ],
        out_specs=pl.BlockSpec((tm, 2 * D), lambda i: (i, 0)),
        out_shape=jax.ShapeDtypeStruct((xp.shape[0], 2 * D), jnp.float32),
    )(xp, wa, wb)
    return out[:n]


def _matmul_res_body(x_ref, w_ref, r_ref, o_ref):
    o_ref[...] = (
        jnp.dot(x_ref[...], w_ref[...], preferred_element_type=jnp.float32)
        + r_ref[...])


def _matmul_res(x, w, res, tm=1024):
    n = x.shape[0]
    xp = _pad_rows(x, tm)
    rp = _pad_rows(res, tm)
    g = xp.shape[0] // tm
    out = pl.pallas_call(
        _matmul_res_body,
        grid=(g,),
        in_specs=[
            pl.BlockSpec((tm, D), lambda i: (i, 0)),
            pl.BlockSpec((D, D), lambda i: (0, 0)),
            pl.BlockSpec((tm, D), lambda i: (i, 0)),
        ],
        out_specs=pl.BlockSpec((tm, D), lambda i: (i, 0)),
        out_shape=jax.ShapeDtypeStruct((xp.shape[0], D), jnp.float32),
    )(xp, w, rp)
    return out[:n]


def _wrap(a):
    return (a + jnp.pi) % (2.0 * jnp.pi) - jnp.pi


def _edge_geom(gs, gd):
    dx = gs[:, 0] - gd[:, 0]
    dy = gs[:, 1] - gd[:, 1]
    h_dst = gd[:, 2]
    c = jnp.cos(h_dst)
    s = jnp.sin(h_dst)
    vx = c * dx + s * dy
    vy = -s * dx + c * dy
    ln = jnp.sqrt(vx * vx + vy * vy + 1e-12)
    th = jnp.arctan2(vy, vx)
    hd = _wrap(gs[:, 2] - h_dst)
    return ln, th, hd


def _pad_cols(x, k):
    return jnp.pad(x, ((0, 0), (0, k - x.shape[1])))


def _attn_layer(x_src, x_dst, src, dst, kattr, vattr, p):
    E = src.shape[0]
    n = x_dst.shape[0]
    q = _matmul1(x_dst, p['Wq'])
    kv = _matmulkv(x_src, p['Wk'], p['Wv'])
    qd = _sc_gather(q.astype(jnp.bfloat16), dst).astype(jnp.float32)
    kvg = _sc_gather(kv.astype(jnp.bfloat16), src).astype(jnp.float32)
    ke = kvg[:, :D] + kattr
    ve = kvg[:, D:] + vattr
    sc = jnp.sum((qd * ke).reshape(E, H, DH), axis=-1) / jnp.sqrt(1.0 * DH)
    e = jnp.exp(sc - jnp.max(sc))
    ev = jnp.concatenate(
        [e, (e[..., None] * ve.reshape(E, H, DH)).reshape(E, D)], axis=1)
    u = jax.ops.segment_sum(ev, dst, num_segments=n)
    agg = (u[:, H:].reshape(n, H, DH)
           / (u[:, :H, None] + 1e-16)).reshape(n, D)
    return _matmul_res(agg, p['Wo'], x_dst)


def kernel(centerline_feats, centerline_heading, centerline_length,
           centerline_position, lane_feats, lane_heading, lane_length,
           lane_position, lane_is_intersection, lane_turn_direction,
           lane_traffic_control, centerline_to_lane_edge_index,
           adjacent_edge_index, predecessor_edge_index, successor_edge_index,
           params):
    prm = params

    c_input = jnp.concatenate(
        [centerline_feats, centerline_heading[:, None], centerline_length[:, None]],
        axis=-1)
    c_embs = _mlp1(_pad_cols(c_input, 8),
                   _pad_cols(prm['c_emb']['W1'].T, 8).T, prm['c_emb']['b1'],
                   prm['c_emb']['W2'], prm['c_emb']['b2'])
    l_input = jnp.concatenate(
        [lane_feats, lane_length[:, None], lane_heading[:, None],
         lane_is_intersection[:, None], lane_turn_direction[:, None],
         lane_traffic_control[:, None]], axis=-1)
    l_embs = _mlp1(_pad_cols(l_input, 8),
                   _pad_cols(prm['l_emb']['W1'].T, 8).T, prm['l_emb']['b1'],
                   prm['l_emb']['W2'], prm['l_emb']['b2'])

    nc = centerline_heading.shape[0]
    zc = jnp.zeros((nc, 13), jnp.float32)
    zl = jnp.zeros((NL, 13), jnp.float32)
    cattr = jnp.concatenate(
        [centerline_position, centerline_heading[:, None], zc], axis=1)
    lattr = jnp.concatenate(
        [lane_position, lane_heading[:, None], zl], axis=1)
    cl_table = jnp.concatenate([cattr, lattr], axis=0)

    c2l_src = centerline_to_lane_edge_index[0]
    c2l_dst = centerline_to_lane_edge_index[1]
    ec = c2l_src.shape[0]
    g = _sc_gather(cl_table,
                   jnp.concatenate([c2l_src, c2l_dst + nc]).astype(jnp.int32))
    ln, th, hd = _edge_geom(g[:ec], g[ec:])
    c2l_feats = jnp.stack([ln, th, hd], axis=-1)
    pe, pa = prm['c2l_emb'], prm['c2l_attn']
    kattr_c, vattr_c = _mlp2(
        _pad_cols(c2l_feats, 8),
        _pad_cols(pe['W1'].T, 8).T, pe['b1'],
        pe['W2'] @ pa['Wek'], pe['b2'] @ pa['Wek'],
        pe['W2'] @ pa['Wev'], pe['b2'] @ pa['Wev'])

    psrc2, ssrc2 = _sc_nhop(predecessor_edge_index, successor_edge_index)
    pdt = predecessor_edge_index.dtype
    pred2 = jnp.stack([psrc2.astype(pdt), predecessor_edge_index[1]])
    succ2 = jnp.stack([ssrc2.astype(pdt), successor_edge_index[1]])
    edge_list = [adjacent_edge_index, predecessor_edge_index, pred2,
                 successor_edge_index, succ2]
    type_rows = [jnp.array([1.0, 0.0, 0.0]), jnp.array([0.0, 1.0, 0.0]),
                 jnp.array([0.0, 1.0, 0.0]), jnp.array([0.0, 0.0, 1.0]),
                 jnp.array([0.0, 0.0, 1.0])]
    hop_vals = [1.0, 1.0, 2.0, 1.0, 2.0]
    l2l_ei = jnp.concatenate(edge_list, axis=1)
    tp = jnp.concatenate(
        [jnp.tile(r[None], (e.shape[1], 1)) for r, e in zip(type_rows, edge_list)],
        axis=0)
    hp = jnp.concatenate(
        [h * jnp.ones(e.shape[1]) for h, e in zip(hop_vals, edge_list)], axis=0)
    l2l_src, l2l_dst = l2l_ei[0], l2l_ei[1]
    el = l2l_src.shape[0]
    g2 = _sc_gather(lattr,
                    jnp.concatenate([l2l_src, l2l_dst]).astype(jnp.int32))
    ln2, th2, hd2 = _edge_geom(g2[:el], g2[el:])
    l2l_feats = jnp.concatenate(
        [ln2[:, None], th2[:, None], hd2[:, None], hp[:, None], tp], axis=-1)
    pe2, pa2 = prm['l2l_emb'], prm['l2l_attn']
    kattr_l, vattr_l = _mlp2(
        _pad_cols(l2l_feats, 8),
        _pad_cols(pe2['W1'].T, 8).T, pe2['b1'],
        pe2['W2'] @ pa2['Wek'], pe2['b2'] @ pa2['Wek'],
        pe2['W2'] @ pa2['Wev'], pe2['b2'] @ pa2['Wev'])

    l_embs = _attn_layer(c_embs, l_embs, c2l_src, c2l_dst, kattr_c, vattr_c,
                         prm['c2l_attn'])
    l_embs = _attn_layer(l_embs, l_embs, l2l_src, l2l_dst, kattr_l, vattr_l,
                         prm['l2l_attn'])
    return l_embs[None]


# speedup vs baseline: 1.1086x; 1.1086x over previous
import functools

import jax
import jax.numpy as jnp
from jax import lax
from jax.experimental import pallas as pl
from jax.experimental.pallas import tpu as pltpu
from jax.experimental.pallas import tpu_sc as plsc

D = 128
H = 8
DH = D // H
NL = 50000
HOPS = 2


def _pad_rows(x, tm):
    n = x.shape[0]
    npad = ((n + tm - 1) // tm) * tm
    if npad != n:
        x = jnp.pad(x, ((0, npad - n),) + ((0, 0),) * (x.ndim - 1))
    return x


_SC_NW = 32
_SC_CH = 128


def _gather_k(C):
    return 8 if C <= 32 else (4 if C <= 128 else 2)


@functools.cache
def _sc_gather_call(Ep, C, dtype_name):
    chunks = Ep // (_SC_NW * _SC_CH)
    K = _gather_k(C)
    dtype = jnp.dtype(dtype_name)
    mesh = plsc.VectorSubcoreMesh(core_axis_name="c", subcore_axis_name="s")
    cparams = (pltpu.CompilerParams(use_tc_tiling_on_sc=False)
               if C % 128 else None)

    @functools.partial(
        pl.kernel, mesh=mesh,
        out_type=jax.ShapeDtypeStruct((Ep, C), dtype),
        compiler_params=cparams,
        scratch_types=[
            pltpu.VMEM((chunks, _SC_CH), jnp.int32),
            pltpu.VMEM((K * _SC_CH, C), dtype),
            pltpu.SemaphoreType.DMA,
        ],
    )
    def gk(table_hbm, idx_hbm, out_hbm, idx_v, buf, sem):
        wid = lax.axis_index("s") * 2 + lax.axis_index("c")
        pltpu.sync_copy(idx_hbm.at[wid], idx_v)
        base = wid * chunks * _SC_CH

        def body(g, carry):
            j0 = g * K
            for t in range(K):
                pltpu.async_copy(table_hbm.at[idx_v.at[j0 + t]],
                                 buf.at[pl.ds(t * _SC_CH, _SC_CH)], sem)
            for t in range(K):
                pltpu.make_async_copy(
                    table_hbm.at[idx_v.at[j0 + t]],
                    buf.at[pl.ds(t * _SC_CH, _SC_CH)], sem).wait()
            pltpu.sync_copy(
                buf, out_hbm.at[pl.ds(base + j0 * _SC_CH, K * _SC_CH)])
            return carry

        lax.fori_loop(0, chunks // K, body, 0)

    return gk


def _sc_gather(table, idx):
    E = idx.shape[0]
    C = table.shape[1]
    unit = _SC_NW * _SC_CH * _gather_k(C)
    Ep = ((E + unit - 1) // unit) * unit
    chunks = Ep // (_SC_NW * _SC_CH)
    idxp = jnp.pad(idx.astype(jnp.int32), (0, Ep - E)).reshape(_SC_NW, chunks, _SC_CH)
    out = _sc_gather_call(Ep, C, str(table.dtype))(table, idxp)
    return out[:E]


_NH_CH = 8192


@functools.cache
def _sc_nhop_call(Ep_p, Es_p):
    n_pad = NL + 16
    mesh = plsc.VectorSubcoreMesh(core_axis_name="c", subcore_axis_name="s")

    @functools.partial(
        pl.kernel, mesh=mesh,
        out_type=(jax.ShapeDtypeStruct((Ep_p,), jnp.int32),
                  jax.ShapeDtypeStruct((Es_p,), jnp.int32)),
        compiler_params=pltpu.CompilerParams(needs_layout_passes=False),
        scratch_types=[
            pltpu.VMEM((n_pad,), jnp.int32),
            pltpu.VMEM((_NH_CH,), jnp.int32),
            pltpu.VMEM((_NH_CH,), jnp.int32),
            pltpu.VMEM((_NH_CH,), jnp.int32),
        ],
    )
    def nk(pdst_hbm, psrc_hbm, sdst_hbm, ssrc_hbm, pout_hbm, sout_hbm,
           m_v, dst_v, src_v, out_v):
        wid = lax.axis_index("s") * 2 + lax.axis_index("c")

        iota16 = jnp.arange(16, dtype=jnp.int32)

        def run(dst_hbm, src_hbm, out_hbm, e_pad):
            nchunk = e_pad // _NH_CH

            def init(k, c):
                m_v[pl.ds(k * 16, 16)] = iota16 + k * 16
                return c

            lax.fori_loop(0, n_pad // 16, init, 0)

            def chunk(j, c):
                pltpu.sync_copy(dst_hbm.at[pl.ds(j * _NH_CH, _NH_CH)], dst_v)
                pltpu.sync_copy(src_hbm.at[pl.ds(j * _NH_CH, _NH_CH)], src_v)

                def upd16(k, c2):
                    dv = dst_v[pl.ds(k * 16, 16)]
                    sv = src_v[pl.ds(k * 16, 16)]
                    p = (iota16 << 16) | sv
                    plsc.store_scatter(m_v, [dv], p)

                    def rnd(r, c3):
                        gw = plsc.load_gather(m_v, [dv])
                        plsc.store_scatter(m_v, [dv], p, mask=p > gw)
                        return c3

                    lax.fori_loop(0, 2, rnd, 0)
                    return c2

                lax.fori_loop(0, _NH_CH // 16, upd16, 0)
                return c

            lax.fori_loop(0, nchunk, chunk, 0)

            def chunk2(j, c):
                pltpu.sync_copy(src_hbm.at[pl.ds(j * _NH_CH, _NH_CH)], src_v)

                def gat(k, c2):
                    idx = src_v[pl.ds(k * 16, 16)]
                    g = plsc.load_gather(m_v, [idx])
                    out_v[pl.ds(k * 16, 16)] = g & 0xFFFF
                    return c2

                lax.fori_loop(0, _NH_CH // 16, gat, 0, unroll=4)
                pltpu.sync_copy(out_v, out_hbm.at[pl.ds(j * _NH_CH, _NH_CH)])
                return c

            lax.fori_loop(0, nchunk, chunk2, 0)

        @pl.when(wid == 0)
        def _():
            run(pdst_hbm, psrc_hbm, pout_hbm, Ep_p)

        @pl.when(wid == 1)
        def _():
            run(sdst_hbm, ssrc_hbm, sout_hbm, Es_p)

    return nk


def _sc_nhop(pred_ei, succ_ei):
    ep, es = pred_ei.shape[1], succ_ei.shape[1]
    epp = ((ep + _NH_CH - 1) // _NH_CH) * _NH_CH
    esp = ((es + _NH_CH - 1) // _NH_CH) * _NH_CH

    def prep(ei, n_pad_to):
        dst = jnp.pad(ei[1].astype(jnp.int32), (0, n_pad_to - ei.shape[1]),
                      constant_values=NL)
        src = jnp.pad(ei[0].astype(jnp.int32), (0, n_pad_to - ei.shape[1]))
        return dst, src

    pdst, psrc = prep(pred_ei, epp)
    sdst, ssrc = prep(succ_ei, esp)
    pout, sout = _sc_nhop_call(epp, esp)(pdst, psrc, sdst, ssrc)
    return pout[:ep], sout[:es]


def _mlp1_body(x_ref, w1_ref, b1_ref, w2_ref, b2_ref, o_ref):
    h = jnp.maximum(
        jnp.dot(x_ref[...], w1_ref[...], preferred_element_type=jnp.float32)
        + b1_ref[...], 0.0)
    o_ref[...] = jnp.dot(h, w2_ref[...], preferred_element_type=jnp.float32) + b2_ref[...]


def _mlp1(x, w1, b1, w2, b2, tm=1024):
    n, din = x.shape
    xp = _pad_rows(x, tm)
    g = xp.shape[0] // tm
    out = pl.pallas_call(
        _mlp1_body,
        grid=(g,),
        in_specs=[
            pl.BlockSpec((tm, din), lambda i: (i, 0)),
            pl.BlockSpec((din, D), lambda i: (0, 0)),
            pl.BlockSpec((1, D), lambda i: (0, 0)),
            pl.BlockSpec((D, D), lambda i: (0, 0)),
            pl.BlockSpec((1, D), lambda i: (0, 0)),
        ],
        out_specs=pl.BlockSpec((tm, D), lambda i: (i, 0)),
        out_shape=jax.ShapeDtypeStruct((xp.shape[0], D), jnp.float32),
    )(xp, w1, b1.reshape(1, D), w2, b2.reshape(1, D))
    return out[:n]


def _mlp2_body(x_ref, w1_ref, b1_ref, wa_ref, ba_ref, wb_ref, bb_ref, oa_ref, ob_ref):
    h = jnp.maximum(
        jnp.dot(x_ref[...], w1_ref[...], preferred_element_type=jnp.float32)
        + b1_ref[...], 0.0)
    oa_ref[...] = jnp.dot(h, wa_ref[...], preferred_element_type=jnp.float32) + ba_ref[...]
    ob_ref[...] = jnp.dot(h, wb_ref[...], preferred_element_type=jnp.float32) + bb_ref[...]


def _mlp2(x, w1, b1, wa, ba, wb, bb, tm=1024):
    n, din = x.shape
    xp = _pad_rows(x, tm)
    g = xp.shape[0] // tm
    oa, ob = pl.pallas_call(
        _mlp2_body,
        grid=(g,),
        in_specs=[
            pl.BlockSpec((tm, din), lambda i: (i, 0)),
            pl.BlockSpec((din, D), lambda i: (0, 0)),
            pl.BlockSpec((1, D), lambda i: (0, 0)),
            pl.BlockSpec((D, D), lambda i: (0, 0)),
            pl.BlockSpec((1, D), lambda i: (0, 0)),
            pl.BlockSpec((D, D), lambda i: (0, 0)),
            pl.BlockSpec((1, D), lambda i: (0, 0)),
        ],
        out_specs=[pl.BlockSpec((tm, D), lambda i: (i, 0))] * 2,
        out_shape=[jax.ShapeDtypeStruct((xp.shape[0], D), jnp.float32)] * 2,
    )(xp, w1, b1.reshape(1, D), wa, ba.reshape(1, D), wb, bb.reshape(1, D))
    return oa[:n], ob[:n]


def _matmul1_body(x_ref, w_ref, o_ref):
    o_ref[...] = jnp.dot(x_ref[...], w_ref[...], preferred_element_type=jnp.float32)


def _matmul1(x, w, tm=1024):
    n = x.shape[0]
    xp = _pad_rows(x, tm)
    g = xp.shape[0] // tm
    out = pl.pallas_call(
        _matmul1_body,
        grid=(g,),
        in_specs=[
            pl.BlockSpec((tm, D), lambda i: (i, 0)),
            pl.BlockSpec((D, D), lambda i: (0, 0)),
        ],
        out_specs=pl.BlockSpec((tm, D), lambda i: (i, 0)),
        out_shape=jax.ShapeDtypeStruct((xp.shape[0], D), jnp.float32),
    )(xp, w)
    return out[:n]


def _matmulkv_body(x_ref, wa_ref, wb_ref, o_ref):
    x = x_ref[...]
    o_ref[:, :D] = jnp.dot(x, wa_ref[...], preferred_element_type=jnp.float32)
    o_ref[:, D:] = jnp.dot(x, wb_ref[...], preferred_element_type=jnp.float32)


def _matmulkv(x, wa, wb, tm=1024):
    n = x.shape[0]
    xp = _pad_rows(x, tm)
    g = xp.shape[0] // tm
    out = pl.pallas_call(
        _matmulkv_body,
        grid=(g,),
        in_specs=[
            pl.BlockSpec((tm, D), lambda i: (i, 0)),
            pl.BlockSpec((D, D), lambda i: (0, 0)),
            pl.BlockSpec((D, D), lambda i: (0, 0)),
        ],
        out_specs=pl.BlockSpec((tm, 2 * D), lambda i: (i, 0)),
        out_shape=jax.ShapeDtypeStruct((xp.shape[0], 2 * D), jnp.float32),
    )(xp, wa, wb)
    return out[:n]


def _matmul_res_body(x_ref, w_ref, r_ref, o_ref):
    o_ref[...] = (
        jnp.dot(x_ref[...], w_ref[...], preferred_element_type=jnp.float32)
        + r_ref[...])


def _matmul_res(x, w, res, tm=1024):
    n = x.shape[0]
    xp = _pad_rows(x, tm)
    rp = _pad_rows(res, tm)
    g = xp.shape[0] // tm
    out = pl.pallas_call(
        _matmul_res_body,
        grid=(g,),
        in_specs=[
            pl.BlockSpec((tm, D), lambda i: (i, 0)),
            pl.BlockSpec((D, D), lambda i: (0, 0)),
            pl.BlockSpec((tm, D), lambda i: (i, 0)),
        ],
        out_specs=pl.BlockSpec((tm, D), lambda i: (i, 0)),
        out_shape=jax.ShapeDtypeStruct((xp.shape[0], D), jnp.float32),
    )(xp, w, rp)
    return out[:n]


def _wrap(a):
    return (a + jnp.pi) % (2.0 * jnp.pi) - jnp.pi


def _edge_geom(gs, gd):
    dx = gs[:, 0] - gd[:, 0]
    dy = gs[:, 1] - gd[:, 1]
    h_dst = gd[:, 2]
    c = jnp.cos(h_dst)
    s = jnp.sin(h_dst)
    vx = c * dx + s * dy
    vy = -s * dx + c * dy
    ln = jnp.sqrt(vx * vx + vy * vy + 1e-12)
    th = jnp.arctan2(vy, vx)
    hd = _wrap(gs[:, 2] - h_dst)
    return ln, th, hd


def _pad_cols(x, k):
    return jnp.pad(x, ((0, 0), (0, k - x.shape[1])))


def _attn_layer(x_src, x_dst, src, dst, kattr, vattr, p):
    E = src.shape[0]
    n = x_dst.shape[0]
    q = _matmul1(x_dst, p['Wq'])
    kv = _matmulkv(x_src, p['Wk'], p['Wv'])
    qd = _sc_gather(q, dst)
    kvg = _sc_gather(kv, src)
    ke = kvg[:, :D] + kattr
    ve = kvg[:, D:] + vattr
    sc = jnp.sum((qd * ke).reshape(E, H, DH), axis=-1) / jnp.sqrt(1.0 * DH)
    e = jnp.exp(sc - jnp.max(sc))
    ev = jnp.concatenate(
        [e, (e[..., None] * ve.reshape(E, H, DH)).reshape(E, D)], axis=1)
    u = jax.ops.segment_sum(ev, dst, num_segments=n)
    agg = (u[:, H:].reshape(n, H, DH)
           / (u[:, :H, None] + 1e-16)).reshape(n, D)
    return _matmul_res(agg, p['Wo'], x_dst)


def kernel(centerline_feats, centerline_heading, centerline_length,
           centerline_position, lane_feats, lane_heading, lane_length,
           lane_position, lane_is_intersection, lane_turn_direction,
           lane_traffic_control, centerline_to_lane_edge_index,
           adjacent_edge_index, predecessor_edge_index, successor_edge_index,
           params):
    prm = params

    c_input = jnp.concatenate(
        [centerline_feats, centerline_heading[:, None], centerline_length[:, None]],
        axis=-1)
    c_embs = _mlp1(_pad_cols(c_input, 8),
                   _pad_cols(prm['c_emb']['W1'].T, 8).T, prm['c_emb']['b1'],
                   prm['c_emb']['W2'], prm['c_emb']['b2'])
    l_input = jnp.concatenate(
        [lane_feats, lane_length[:, None], lane_heading[:, None],
         lane_is_intersection[:, None], lane_turn_direction[:, None],
         lane_traffic_control[:, None]], axis=-1)
    l_embs = _mlp1(_pad_cols(l_input, 8),
                   _pad_cols(prm['l_emb']['W1'].T, 8).T, prm['l_emb']['b1'],
                   prm['l_emb']['W2'], prm['l_emb']['b2'])

    nc = centerline_heading.shape[0]
    zc = jnp.zeros((nc, 13), jnp.float32)
    zl = jnp.zeros((NL, 13), jnp.float32)
    cattr = jnp.concatenate(
        [centerline_position, centerline_heading[:, None], zc], axis=1)
    lattr = jnp.concatenate(
        [lane_position, lane_heading[:, None], zl], axis=1)
    cl_table = jnp.concatenate([cattr, lattr], axis=0)

    c2l_src = centerline_to_lane_edge_index[0]
    c2l_dst = centerline_to_lane_edge_index[1]
    ec = c2l_src.shape[0]
    g = _sc_gather(cl_table,
                   jnp.concatenate([c2l_src, c2l_dst + nc]).astype(jnp.int32))
    ln, th, hd = _edge_geom(g[:ec], g[ec:])
    c2l_feats = jnp.stack([ln, th, hd], axis=-1)
    pe, pa = prm['c2l_emb'], prm['c2l_attn']
    kattr_c, vattr_c = _mlp2(
        _pad_cols(c2l_feats, 8),
        _pad_cols(pe['W1'].T, 8).T, pe['b1'],
        pe['W2'] @ pa['Wek'], pe['b2'] @ pa['Wek'],
        pe['W2'] @ pa['Wev'], pe['b2'] @ pa['Wev'])

    psrc2, ssrc2 = _sc_nhop(predecessor_edge_index, successor_edge_index)
    pdt = predecessor_edge_index.dtype
    pred2 = jnp.stack([psrc2.astype(pdt), predecessor_edge_index[1]])
    succ2 = jnp.stack([ssrc2.astype(pdt), successor_edge_index[1]])
    edge_list = [adjacent_edge_index, predecessor_edge_index, pred2,
                 successor_edge_index, succ2]
    type_rows = [jnp.array([1.0, 0.0, 0.0]), jnp.array([0.0, 1.0, 0.0]),
                 jnp.array([0.0, 1.0, 0.0]), jnp.array([0.0, 0.0, 1.0]),
                 jnp.array([0.0, 0.0, 1.0])]
    hop_vals = [1.0, 1.0, 2.0, 1.0, 2.0]
    l2l_ei = jnp.concatenate(edge_list, axis=1)
    tp = jnp.concatenate(
        [jnp.tile(r[None], (e.shape[1], 1)) for r, e in zip(type_rows, edge_list)],
        axis=0)
    hp = jnp.concatenate(
        [h * jnp.ones(e.shape[1]) for h, e in zip(hop_vals, edge_list)], axis=0)
    l2l_src, l2l_dst = l2l_ei[0], l2l_ei[1]
    el = l2l_src.shape[0]
    g2 = _sc_gather(lattr,
                    jnp.concatenate([l2l_src, l2l_dst]).astype(jnp.int32))
    ln2, th2, hd2 = _edge_geom(g2[:el], g2[el:])
    l2l_feats = jnp.concatenate(
        [ln2[:, None], th2[:, None], hd2[:, None], hp[:, None], tp], axis=-1)
    pe2, pa2 = prm['l2l_emb'], prm['l2l_attn']
    kattr_l, vattr_l = _mlp2(
        _pad_cols(l2l_feats, 8),
        _pad_cols(pe2['W1'].T, 8).T, pe2['b1'],
        pe2['W2'] @ pa2['Wek'], pe2['b2'] @ pa2['Wek'],
        pe2['W2'] @ pa2['Wev'], pe2['b2'] @ pa2['Wev'])

    l_embs = _attn_layer(c_embs, l_embs, c2l_src, c2l_dst, kattr_c, vattr_c,
                         prm['c2l_attn'])
    l_embs = _attn_layer(l_embs, l_embs, l2l_src, l2l_dst, kattr_l, vattr_l,
                         prm['l2l_attn'])
    return l_embs[None]


# speedup vs baseline: 1.1567x; 1.0434x over previous
import functools

import jax
import jax.numpy as jnp
from jax import lax
from jax.experimental import pallas as pl
from jax.experimental.pallas import tpu as pltpu
from jax.experimental.pallas import tpu_sc as plsc

D = 128
H = 8
DH = D // H
NL = 50000
HOPS = 2


def _pad_rows(x, tm):
    n = x.shape[0]
    npad = ((n + tm - 1) // tm) * tm
    if npad != n:
        x = jnp.pad(x, ((0, npad - n),) + ((0, 0),) * (x.ndim - 1))
    return x


_SC_NW = 32
_SC_CH = 128


@functools.cache
def _sc_gather_call(Ep, C, dtype_name):
    chunks = Ep // (_SC_NW * _SC_CH)
    dtype = jnp.dtype(dtype_name)
    mesh = plsc.VectorSubcoreMesh(core_axis_name="c", subcore_axis_name="s")
    cparams = (pltpu.CompilerParams(use_tc_tiling_on_sc=False)
               if C % 128 else None)

    @functools.partial(
        pl.kernel, mesh=mesh,
        out_type=jax.ShapeDtypeStruct((Ep, C), dtype),
        compiler_params=cparams,
        scratch_types=[
            pltpu.VMEM((chunks, _SC_CH), jnp.int32),
            pltpu.VMEM((_SC_CH, C), dtype),
            pltpu.SemaphoreType.DMA,
        ],
    )
    def gk(table_hbm, idx_hbm, out_hbm, idx_v, buf, sem):
        wid = lax.axis_index("s") * 2 + lax.axis_index("c")
        pltpu.sync_copy(idx_hbm.at[wid], idx_v)
        base = wid * chunks * _SC_CH

        def body(j, carry):
            pltpu.async_copy(table_hbm.at[idx_v.at[j]], buf, sem).wait()
            pltpu.sync_copy(buf, out_hbm.at[pl.ds(base + j * _SC_CH, _SC_CH)])
            return carry

        lax.fori_loop(0, chunks, body, 0)

    return gk


def _sc_gather(table, idx):
    E = idx.shape[0]
    C = table.shape[1]
    unit = _SC_NW * _SC_CH
    Ep = ((E + unit - 1) // unit) * unit
    chunks = Ep // unit
    idxp = jnp.pad(idx.astype(jnp.int32), (0, Ep - E)).reshape(_SC_NW, chunks, _SC_CH)
    out = _sc_gather_call(Ep, C, str(table.dtype))(table, idxp)
    return out[:E]


@functools.cache
def _sc_gather2_call(Ep):
    chunks = Ep // (_SC_NW * _SC_CH)
    mesh = plsc.VectorSubcoreMesh(core_axis_name="c", subcore_axis_name="s")

    @functools.partial(
        pl.kernel, mesh=mesh,
        out_type=(jax.ShapeDtypeStruct((Ep, D), jnp.float32),
                  jax.ShapeDtypeStruct((Ep, 2 * D), jnp.float32)),
        scratch_types=[
            pltpu.VMEM((chunks, _SC_CH), jnp.int32),
            pltpu.VMEM((chunks, _SC_CH), jnp.int32),
            pltpu.VMEM((_SC_CH, D), jnp.float32),
            pltpu.VMEM((_SC_CH, 2 * D), jnp.float32),
            pltpu.SemaphoreType.DMA,
            pltpu.SemaphoreType.DMA,
        ],
    )
    def gk(qt_hbm, kvt_hbm, idxq_hbm, idxk_hbm, oq_hbm, okv_hbm,
           iq_v, ik_v, bq, bk, sq, sk):
        wid = lax.axis_index("s") * 2 + lax.axis_index("c")
        pltpu.sync_copy(idxq_hbm.at[wid], iq_v)
        pltpu.sync_copy(idxk_hbm.at[wid], ik_v)
        base = wid * chunks * _SC_CH

        def body(j, carry):
            cq = pltpu.async_copy(qt_hbm.at[iq_v.at[j]], bq, sq)
            ck = pltpu.async_copy(kvt_hbm.at[ik_v.at[j]], bk, sk)
            cq.wait()
            pltpu.sync_copy(bq, oq_hbm.at[pl.ds(base + j * _SC_CH, _SC_CH)])
            ck.wait()
            pltpu.sync_copy(bk, okv_hbm.at[pl.ds(base + j * _SC_CH, _SC_CH)])
            return carry

        lax.fori_loop(0, chunks, body, 0)

    return gk


def _sc_gather_qkv(q, kv, dst, src):
    E = dst.shape[0]
    unit = _SC_NW * _SC_CH
    Ep = ((E + unit - 1) // unit) * unit
    chunks = Ep // unit

    def prep(idx):
        return jnp.pad(idx.astype(jnp.int32), (0, Ep - E)).reshape(
            _SC_NW, chunks, _SC_CH)

    oq, okv = _sc_gather2_call(Ep)(q, kv, prep(dst), prep(src))
    return oq[:E], okv[:E]


_NH_CH = 8192


@functools.cache
def _sc_nhop_call(Ep_p, Es_p):
    n_pad = NL + 16
    mesh = plsc.VectorSubcoreMesh(core_axis_name="c", subcore_axis_name="s")

    @functools.partial(
        pl.kernel, mesh=mesh,
        out_type=(jax.ShapeDtypeStruct((Ep_p,), jnp.int32),
                  jax.ShapeDtypeStruct((Es_p,), jnp.int32)),
        compiler_params=pltpu.CompilerParams(needs_layout_passes=False),
        scratch_types=[
            pltpu.VMEM((n_pad,), jnp.int32),
            pltpu.VMEM((_NH_CH,), jnp.int32),
            pltpu.VMEM((_NH_CH,), jnp.int32),
            pltpu.VMEM((_NH_CH,), jnp.int32),
        ],
    )
    def nk(pdst_hbm, psrc_hbm, sdst_hbm, ssrc_hbm, pout_hbm, sout_hbm,
           m_v, dst_v, src_v, out_v):
        wid = lax.axis_index("s") * 2 + lax.axis_index("c")

        iota16 = jnp.arange(16, dtype=jnp.int32)

        def run(dst_hbm, src_hbm, out_hbm, e_pad):
            nchunk = e_pad // _NH_CH

            def init(k, c):
                m_v[pl.ds(k * 16, 16)] = iota16 + k * 16
                return c

            lax.fori_loop(0, n_pad // 16, init, 0)

            def chunk(j, c):
                pltpu.sync_copy(dst_hbm.at[pl.ds(j * _NH_CH, _NH_CH)], dst_v)
                pltpu.sync_copy(src_hbm.at[pl.ds(j * _NH_CH, _NH_CH)], src_v)

                def upd16(k, c2):
                    dv = dst_v[pl.ds(k * 16, 16)]
                    sv = src_v[pl.ds(k * 16, 16)]
                    p = (iota16 << 16) | sv
                    plsc.store_scatter(m_v, [dv], p)

                    def rnd(r, c3):
                        gw = plsc.load_gather(m_v, [dv])
                        plsc.store_scatter(m_v, [dv], p, mask=p > gw)
                        return c3

                    lax.fori_loop(0, 2, rnd, 0)
                    return c2

                lax.fori_loop(0, _NH_CH // 16, upd16, 0)
                return c

            lax.fori_loop(0, nchunk, chunk, 0)

            def chunk2(j, c):
                pltpu.sync_copy(src_hbm.at[pl.ds(j * _NH_CH, _NH_CH)], src_v)

                def gat(k, c2):
                    idx = src_v[pl.ds(k * 16, 16)]
                    g = plsc.load_gather(m_v, [idx])
                    out_v[pl.ds(k * 16, 16)] = g & 0xFFFF
                    return c2

                lax.fori_loop(0, _NH_CH // 16, gat, 0, unroll=4)
                pltpu.sync_copy(out_v, out_hbm.at[pl.ds(j * _NH_CH, _NH_CH)])
                return c

            lax.fori_loop(0, nchunk, chunk2, 0)

        @pl.when(wid == 0)
        def _():
            run(pdst_hbm, psrc_hbm, pout_hbm, Ep_p)

        @pl.when(wid == 1)
        def _():
            run(sdst_hbm, ssrc_hbm, sout_hbm, Es_p)

    return nk


def _sc_nhop(pred_ei, succ_ei):
    ep, es = pred_ei.shape[1], succ_ei.shape[1]
    epp = ((ep + _NH_CH - 1) // _NH_CH) * _NH_CH
    esp = ((es + _NH_CH - 1) // _NH_CH) * _NH_CH

    def prep(ei, n_pad_to):
        dst = jnp.pad(ei[1].astype(jnp.int32), (0, n_pad_to - ei.shape[1]),
                      constant_values=NL)
        src = jnp.pad(ei[0].astype(jnp.int32), (0, n_pad_to - ei.shape[1]))
        return dst, src

    pdst, psrc = prep(pred_ei, epp)
    sdst, ssrc = prep(succ_ei, esp)
    pout, sout = _sc_nhop_call(epp, esp)(pdst, psrc, sdst, ssrc)
    return pout[:ep], sout[:es]


def _mlp1_body(x_ref, w1_ref, b1_ref, w2_ref, b2_ref, o_ref):
    h = jnp.maximum(
        jnp.dot(x_ref[...], w1_ref[...], preferred_element_type=jnp.float32)
        + b1_ref[...], 0.0)
    o_ref[...] = jnp.dot(h, w2_ref[...], preferred_element_type=jnp.float32) + b2_ref[...]


def _mlp1(x, w1, b1, w2, b2, tm=1024):
    n, din = x.shape
    xp = _pad_rows(x, tm)
    g = xp.shape[0] // tm
    out = pl.pallas_call(
        _mlp1_body,
        grid=(g,),
        in_specs=[
            pl.BlockSpec((tm, din), lambda i: (i, 0)),
            pl.BlockSpec((din, D), lambda i: (0, 0)),
            pl.BlockSpec((1, D), lambda i: (0, 0)),
            pl.BlockSpec((D, D), lambda i: (0, 0)),
            pl.BlockSpec((1, D), lambda i: (0, 0)),
        ],
        out_specs=pl.BlockSpec((tm, D), lambda i: (i, 0)),
        out_shape=jax.ShapeDtypeStruct((xp.shape[0], D), jnp.float32),
    )(xp, w1, b1.reshape(1, D), w2, b2.reshape(1, D))
    return out[:n]


def _mlp2_body(x_ref, w1_ref, b1_ref, wa_ref, ba_ref, wb_ref, bb_ref, oa_ref, ob_ref):
    h = jnp.maximum(
        jnp.dot(x_ref[...], w1_ref[...], preferred_element_type=jnp.float32)
        + b1_ref[...], 0.0)
    oa_ref[...] = jnp.dot(h, wa_ref[...], preferred_element_type=jnp.float32) + ba_ref[...]
    ob_ref[...] = jnp.dot(h, wb_ref[...], preferred_element_type=jnp.float32) + bb_ref[...]


def _mlp2(x, w1, b1, wa, ba, wb, bb, tm=1024):
    n, din = x.shape
    xp = _pad_rows(x, tm)
    g = xp.shape[0] // tm
    oa, ob = pl.pallas_call(
        _mlp2_body,
        grid=(g,),
        in_specs=[
            pl.BlockSpec((tm, din), lambda i: (i, 0)),
            pl.BlockSpec((din, D), lambda i: (0, 0)),
            pl.BlockSpec((1, D), lambda i: (0, 0)),
            pl.BlockSpec((D, D), lambda i: (0, 0)),
            pl.BlockSpec((1, D), lambda i: (0, 0)),
            pl.BlockSpec((D, D), lambda i: (0, 0)),
            pl.BlockSpec((1, D), lambda i: (0, 0)),
        ],
        out_specs=[pl.BlockSpec((tm, D), lambda i: (i, 0))] * 2,
        out_shape=[jax.ShapeDtypeStruct((xp.shape[0], D), jnp.float32)] * 2,
    )(xp, w1, b1.reshape(1, D), wa, ba.reshape(1, D), wb, bb.reshape(1, D))
    return oa[:n], ob[:n]


def _matmul1_body(x_ref, w_ref, o_ref):
    o_ref[...] = jnp.dot(x_ref[...], w_ref[...], preferred_element_type=jnp.float32)


def _matmul1(x, w, tm=1024):
    n = x.shape[0]
    xp = _pad_rows(x, tm)
    g = xp.shape[0] // tm
    out = pl.pallas_call(
        _matmul1_body,
        grid=(g,),
        in_specs=[
            pl.BlockSpec((tm, D), lambda i: (i, 0)),
            pl.BlockSpec((D, D), lambda i: (0, 0)),
        ],
        out_specs=pl.BlockSpec((tm, D), lambda i: (i, 0)),
        out_shape=jax.ShapeDtypeStruct((xp.shape[0], D), jnp.float32),
    )(xp, w)
    return out[:n]


def _matmulkv_body(x_ref, wa_ref, wb_ref, o_ref):
    x = x_ref[...]
    o_ref[:, :D] = jnp.dot(x, wa_ref[...], preferred_element_type=jnp.float32)
    o_ref[:, D:] = jnp.dot(x, wb_ref[...], preferred_element_type=jnp.float32)


def _matmulkv(x, wa, wb, tm=1024):
    n = x.shape[0]
    xp = _pad_rows(x, tm)
    g = xp.shape[0] // tm
    out = pl.pallas_call(
        _matmulkv_body,
        grid=(g,),
        in_specs=[
            pl.BlockSpec((tm, D), lambda i: (i, 0)),
            pl.BlockSpec((D, D), lambda i: (0, 0)),
            pl.BlockSpec((D, D), lambda i: (0, 0)),
        ],
        out_specs=pl.BlockSpec((tm, 2 * D), lambda i: (i, 0)),
        out_shape=jax.ShapeDtypeStruct((xp.shape[0], 2 * D), jnp.float32),
    )(xp, wa, wb)
    return out[:n]


def _matmul_res_body(x_ref, w_ref, r_ref, o_ref):
    o_ref[...] = (
        jnp.dot(x_ref[...], w_ref[...], preferred_element_type=jnp.float32)
        + r_ref[...])


def _matmul_res(x, w, res, tm=1024):
    n = x.shape[0]
    xp = _pad_rows(x, tm)
    rp = _pad_rows(res, tm)
    g = xp.shape[0] // tm
    out = pl.pallas_call(
        _matmul_res_body,
        grid=(g,),
        in_specs=[
            pl.BlockSpec((tm, D), lambda i: (i, 0)),
            pl.BlockSpec((D, D), lambda i: (0, 0)),
            pl.BlockSpec((tm, D), lambda i: (i, 0)),
        ],
        out_specs=pl.BlockSpec((tm, D), lambda i: (i, 0)),
        out_shape=jax.ShapeDtypeStruct((xp.shape[0], D), jnp.float32),
    )(xp, w, rp)
    return out[:n]


def _wrap(a):
    return (a + jnp.pi) % (2.0 * jnp.pi) - jnp.pi


def _edge_geom(gs, gd):
    dx = gs[:, 0] - gd[:, 0]
    dy = gs[:, 1] - gd[:, 1]
    h_dst = gd[:, 2]
    c = jnp.cos(h_dst)
    s = jnp.sin(h_dst)
    vx = c * dx + s * dy
    vy = -s * dx + c * dy
    ln = jnp.sqrt(vx * vx + vy * vy + 1e-12)
    th = jnp.arctan2(vy, vx)
    hd = _wrap(gs[:, 2] - h_dst)
    return ln, th, hd


def _pad_cols(x, k):
    return jnp.pad(x, ((0, 0), (0, k - x.shape[1])))


def _attn_layer(x_src, x_dst, src, dst, kattr, vattr, p):
    E = src.shape[0]
    n = x_dst.shape[0]
    q = _matmul1(x_dst, p['Wq'])
    kv = _matmulkv(x_src, p['Wk'], p['Wv'])
    qd, kvg = _sc_gather_qkv(q, kv, dst, src)
    ke = kvg[:, :D] + kattr
    ve = kvg[:, D:] + vattr
    sc = jnp.sum((qd * ke).reshape(E, H, DH), axis=-1) / jnp.sqrt(1.0 * DH)
    e = jnp.exp(sc - jnp.max(sc))
    ev = jnp.concatenate(
        [e, (e[..., None] * ve.reshape(E, H, DH)).reshape(E, D)], axis=1)
    u = jax.ops.segment_sum(ev, dst, num_segments=n)
    agg = (u[:, H:].reshape(n, H, DH)
           / (u[:, :H, None] + 1e-16)).reshape(n, D)
    return _matmul_res(agg, p['Wo'], x_dst)


def kernel(centerline_feats, centerline_heading, centerline_length,
           centerline_position, lane_feats, lane_heading, lane_length,
           lane_position, lane_is_intersection, lane_turn_direction,
           lane_traffic_control, centerline_to_lane_edge_index,
           adjacent_edge_index, predecessor_edge_index, successor_edge_index,
           params):
    prm = params

    c_input = jnp.concatenate(
        [centerline_feats, centerline_heading[:, None], centerline_length[:, None]],
        axis=-1)
    c_embs = _mlp1(_pad_cols(c_input, 8),
                   _pad_cols(prm['c_emb']['W1'].T, 8).T, prm['c_emb']['b1'],
                   prm['c_emb']['W2'], prm['c_emb']['b2'])
    l_input = jnp.concatenate(
        [lane_feats, lane_length[:, None], lane_heading[:, None],
         lane_is_intersection[:, None], lane_turn_direction[:, None],
         lane_traffic_control[:, None]], axis=-1)
    l_embs = _mlp1(_pad_cols(l_input, 8),
                   _pad_cols(prm['l_emb']['W1'].T, 8).T, prm['l_emb']['b1'],
                   prm['l_emb']['W2'], prm['l_emb']['b2'])

    nc = centerline_heading.shape[0]
    zc = jnp.zeros((nc, 13), jnp.float32)
    zl = jnp.zeros((NL, 13), jnp.float32)
    cattr = jnp.concatenate(
        [centerline_position, centerline_heading[:, None], zc], axis=1)
    lattr = jnp.concatenate(
        [lane_position, lane_heading[:, None], zl], axis=1)
    cl_table = jnp.concatenate([cattr, lattr], axis=0)

    c2l_src = centerline_to_lane_edge_index[0]
    c2l_dst = centerline_to_lane_edge_index[1]
    ec = c2l_src.shape[0]
    g = _sc_gather(cl_table,
                   jnp.concatenate([c2l_src, c2l_dst + nc]).astype(jnp.int32))
    ln, th, hd = _edge_geom(g[:ec], g[ec:])
    c2l_feats = jnp.stack([ln, th, hd], axis=-1)
    pe, pa = prm['c2l_emb'], prm['c2l_attn']
    kattr_c, vattr_c = _mlp2(
        _pad_cols(c2l_feats, 8),
        _pad_cols(pe['W1'].T, 8).T, pe['b1'],
        pe['W2'] @ pa['Wek'], pe['b2'] @ pa['Wek'],
        pe['W2'] @ pa['Wev'], pe['b2'] @ pa['Wev'])

    psrc2, ssrc2 = _sc_nhop(predecessor_edge_index, successor_edge_index)
    pdt = predecessor_edge_index.dtype
    pred2 = jnp.stack([psrc2.astype(pdt), predecessor_edge_index[1]])
    succ2 = jnp.stack([ssrc2.astype(pdt), successor_edge_index[1]])
    edge_list = [adjacent_edge_index, predecessor_edge_index, pred2,
                 successor_edge_index, succ2]
    type_rows = [jnp.array([1.0, 0.0, 0.0]), jnp.array([0.0, 1.0, 0.0]),
                 jnp.array([0.0, 1.0, 0.0]), jnp.array([0.0, 0.0, 1.0]),
                 jnp.array([0.0, 0.0, 1.0])]
    hop_vals = [1.0, 1.0, 2.0, 1.0, 2.0]
    l2l_ei = jnp.concatenate(edge_list, axis=1)
    tp = jnp.concatenate(
        [jnp.tile(r[None], (e.shape[1], 1)) for r, e in zip(type_rows, edge_list)],
        axis=0)
    hp = jnp.concatenate(
        [h * jnp.ones(e.shape[1]) for h, e in zip(hop_vals, edge_list)], axis=0)
    l2l_src, l2l_dst = l2l_ei[0], l2l_ei[1]
    el = l2l_src.shape[0]
    g2 = _sc_gather(lattr,
                    jnp.concatenate([l2l_src, l2l_dst]).astype(jnp.int32))
    ln2, th2, hd2 = _edge_geom(g2[:el], g2[el:])
    l2l_feats = jnp.concatenate(
        [ln2[:, None], th2[:, None], hd2[:, None], hp[:, None], tp], axis=-1)
    pe2, pa2 = prm['l2l_emb'], prm['l2l_attn']
    kattr_l, vattr_l = _mlp2(
        _pad_cols(l2l_feats, 8),
        _pad_cols(pe2['W1'].T, 8).T, pe2['b1'],
        pe2['W2'] @ pa2['Wek'], pe2['b2'] @ pa2['Wek'],
        pe2['W2'] @ pa2['Wev'], pe2['b2'] @ pa2['Wev'])

    l_embs = _attn_layer(c_embs, l_embs, c2l_src, c2l_dst, kattr_c, vattr_c,
                         prm['c2l_attn'])
    l_embs = _attn_layer(l_embs, l_embs, l2l_src, l2l_dst, kattr_l, vattr_l,
                         prm['l2l_attn'])
    return l_embs[None]
